# baseline (speedup 1.0000x reference)
import functools

import jax
import jax.numpy as jnp
from jax import lax
from jax.experimental import pallas as pl
from jax.experimental.pallas import tpu as pltpu
from jax.experimental.pallas import tpu_sc as plsc

_EMBED_DIM = 32
_BATCH = 16384

_NC = 2
_NS = 16
_NW = _NC * _NS
_BPW = _BATCH // _NW
_CHUNK = 128
_NCHUNK = _BPW // _CHUNK


@jax.jit
def _embedding_lookup_t(tableT, idx):
  mesh = plsc.VectorSubcoreMesh(core_axis_name="c", subcore_axis_name="s")

  @functools.partial(
      pl.kernel,
      mesh=mesh,
      compiler_params=pltpu.CompilerParams(use_tc_tiling_on_sc=False),
      out_type=jax.ShapeDtypeStruct((_EMBED_DIM, _BATCH), jnp.float32),
      scratch_types=[
          pltpu.VMEM((_BPW,), jnp.int32),
          pltpu.VMEM((_EMBED_DIM, _BPW), jnp.float32),
          pltpu.SemaphoreType.DMA,
      ],
  )
  def k(tableT_hbm, idx_hbm, outT_hbm, idx_v, out_v, sem):
    wid = lax.axis_index("s") * _NC + lax.axis_index("c")
    base = wid * _BPW
    pltpu.sync_copy(idx_hbm.at[pl.ds(base, _BPW)], idx_v)
    copies = []
    for c in range(_EMBED_DIM):
      for q in range(_NCHUNK):
        copies.append(
            pltpu.async_copy(
                tableT_hbm.at[c].at[idx_v.at[pl.ds(q * _CHUNK, _CHUNK)]],
                out_v.at[c, pl.ds(q * _CHUNK, _CHUNK)],
                sem,
            ))
    for cp in copies:
      cp.wait()
    pltpu.sync_copy(out_v, outT_hbm.at[:, pl.ds(base, _BPW)])

  return k(tableT, idx)


def kernel(item_fea, table):
  outT = _embedding_lookup_t(table.T, item_fea[:, 0])
  return outT.T
